# baseline (speedup 1.0000x reference)
import functools
import jax
import jax.numpy as jnp
from jax import lax
from jax.experimental import pallas as pl
from jax.experimental.pallas import tpu as pltpu
from jax.experimental.pallas import tpu_sc as plsc

NC, NS = 2, 16
NW = NC * NS
CHUNK = 256
NBUF = 2


def kernel(x, pos_table):
    B, S, D = x.shape
    R = B * S
    rows_per_w = R // NW
    nchunk = rows_per_w // CHUNK
    xf = x.reshape(R, D)
    positions = lax.broadcasted_iota(jnp.int32, (R,), 0) % S

    mesh = plsc.VectorSubcoreMesh(core_axis_name="c", subcore_axis_name="s")

    @functools.partial(
        pl.kernel,
        out_type=jax.ShapeDtypeStruct((R, D), jnp.float32),
        mesh=mesh,
        scratch_types=[
            pltpu.VMEM((rows_per_w,), jnp.int32),
            pltpu.VMEM((NBUF, CHUNK, D), jnp.float32),
            pltpu.SemaphoreType.DMA,
            pltpu.SemaphoreType.DMA,
            pltpu.SemaphoreType.DMA,
        ],
    )
    def sc_add(x_hbm, pos_hbm, pidx_hbm, out_hbm, idx_v, bufs, sem_in, sem_add, sem_out):
        wid = lax.axis_index("s") * NC + lax.axis_index("c")
        base = wid * rows_per_w
        pltpu.sync_copy(pidx_hbm.at[pl.ds(base, rows_per_w)], idx_v)
        for c in range(nchunk):
            b = c % NBUF
            lo = base + c * CHUNK
            pltpu.async_copy(x_hbm.at[pl.ds(lo, CHUNK)], bufs.at[b], sem_in).wait()
            pltpu.async_copy(
                pos_hbm.at[idx_v.at[pl.ds(c * CHUNK, CHUNK)]], bufs.at[b],
                sem_add, add=True,
            ).wait()
            pltpu.async_copy(bufs.at[b], out_hbm.at[pl.ds(lo, CHUNK)], sem_out).wait()

    out = sc_add(xf, pos_table, positions)
    return out.reshape(B, S, D)


# speedup vs baseline: 1.0901x; 1.0901x over previous
import functools
import jax
import jax.numpy as jnp
from jax import lax
from jax.experimental import pallas as pl
from jax.experimental.pallas import tpu as pltpu
from jax.experimental.pallas import tpu_sc as plsc

NC, NS = 2, 16
NW = NC * NS
CHUNK = 128
NBUF = 3


def kernel(x, pos_table):
    B, S, D = x.shape
    R = B * S
    rows_per_w = R // NW
    nchunk = rows_per_w // CHUNK
    xf = x.reshape(R, D)
    positions = lax.broadcasted_iota(jnp.int32, (R,), 0) % S

    mesh = plsc.VectorSubcoreMesh(core_axis_name="c", subcore_axis_name="s")

    @functools.partial(
        pl.kernel,
        out_type=jax.ShapeDtypeStruct((R, D), jnp.float32),
        mesh=mesh,
        scratch_types=[
            pltpu.VMEM((rows_per_w,), jnp.int32),
            pltpu.VMEM((NBUF, CHUNK, D), jnp.float32),
            pltpu.SemaphoreType.DMA((NBUF,)),
            pltpu.SemaphoreType.DMA((NBUF,)),
            pltpu.SemaphoreType.DMA((NBUF,)),
        ],
    )
    def sc_add(x_hbm, pos_hbm, pidx_hbm, out_hbm, idx_v, bufs, sem_in, sem_add, sem_out):
        wid = lax.axis_index("s") * NC + lax.axis_index("c")
        base = wid * rows_per_w
        pltpu.sync_copy(pidx_hbm.at[pl.ds(base, rows_per_w)], idx_v)

        def in_copy(c):
            b = c % NBUF
            return pltpu.async_copy(
                x_hbm.at[pl.ds(base + c * CHUNK, CHUNK)], bufs.at[b], sem_in.at[b])

        def add_copy(c):
            b = c % NBUF
            return pltpu.async_copy(
                pos_hbm.at[idx_v.at[pl.ds(c * CHUNK, CHUNK)]], bufs.at[b],
                sem_add.at[b], add=True)

        def out_copy(c):
            b = c % NBUF
            return pltpu.async_copy(
                bufs.at[b], out_hbm.at[pl.ds(base + c * CHUNK, CHUNK)], sem_out.at[b])

        descs = {}
        for t in range(nchunk + 2):
            if t < nchunk:
                if t >= NBUF:
                    descs[("out", t - NBUF)].wait()
                descs[("in", t)] = in_copy(t)
            c1 = t - 1
            if 0 <= c1 < nchunk:
                descs[("in", c1)].wait()
                descs[("add", c1)] = add_copy(c1)
            c2 = t - 2
            if 0 <= c2 < nchunk:
                descs[("add", c2)].wait()
                descs[("out", c2)] = out_copy(c2)
        for c in range(max(0, nchunk - NBUF), nchunk):
            descs[("out", c)].wait()

    out = sc_add(xf, pos_table, positions)
    return out.reshape(B, S, D)


# speedup vs baseline: 1.0923x; 1.0021x over previous
import functools
import jax
import jax.numpy as jnp
from jax import lax
from jax.experimental import pallas as pl
from jax.experimental.pallas import tpu as pltpu
from jax.experimental.pallas import tpu_sc as plsc

NC, NS = 2, 16
NW = NC * NS
CHUNK = 256
NBUF = 2


def kernel(x, pos_table):
    B, S, D = x.shape
    R = B * S
    rows_per_w = R // NW
    nchunk = rows_per_w // CHUNK
    xf = x.reshape(R, D)
    positions = lax.broadcasted_iota(jnp.int32, (R,), 0) % S

    mesh = plsc.VectorSubcoreMesh(core_axis_name="c", subcore_axis_name="s")

    @functools.partial(
        pl.kernel,
        out_type=jax.ShapeDtypeStruct((R, D), jnp.float32),
        mesh=mesh,
        scratch_types=[
            pltpu.VMEM((rows_per_w,), jnp.int32),
            pltpu.VMEM((NBUF, CHUNK, D), jnp.float32),
            pltpu.SemaphoreType.DMA((NBUF,)),
            pltpu.SemaphoreType.DMA((NBUF,)),
            pltpu.SemaphoreType.DMA((NBUF,)),
        ],
    )
    def sc_add(x_hbm, pos_hbm, pidx_hbm, out_hbm, idx_v, bufs, sem_in, sem_add, sem_out):
        wid = lax.axis_index("s") * NC + lax.axis_index("c")
        base = wid * rows_per_w
        idx_desc = pltpu.async_copy(
            pidx_hbm.at[pl.ds(base, rows_per_w)], idx_v, sem_add.at[0])

        def in_copy(c):
            b = c % NBUF
            return pltpu.async_copy(
                x_hbm.at[pl.ds(base + c * CHUNK, CHUNK)], bufs.at[b], sem_in.at[b])

        def add_copy(c):
            b = c % NBUF
            return pltpu.async_copy(
                pos_hbm.at[idx_v.at[pl.ds(c * CHUNK, CHUNK)]], bufs.at[b],
                sem_add.at[b], add=True)

        def out_copy(c):
            b = c % NBUF
            return pltpu.async_copy(
                bufs.at[b], out_hbm.at[pl.ds(base + c * CHUNK, CHUNK)], sem_out.at[b])

        descs = {}
        descs[("in", 0)] = in_copy(0)
        idx_desc.wait()
        for c in range(nchunk):
            if c + 1 < nchunk:
                if c >= 1:
                    descs[("out", c - 1)].wait()
                descs[("in", c + 1)] = in_copy(c + 1)
            descs[("in", c)].wait()
            descs[("add", c)] = add_copy(c)
            descs[("add", c)].wait()
            descs[("out", c)] = out_copy(c)
        descs[("out", nchunk - 2)].wait()
        descs[("out", nchunk - 1)].wait()

    out = sc_add(xf, pos_table, positions)
    return out.reshape(B, S, D)


# speedup vs baseline: 1.1546x; 1.0570x over previous
import functools
import jax
import jax.numpy as jnp
from jax import lax
from jax.experimental import pallas as pl
from jax.experimental.pallas import tpu as pltpu
from jax.experimental.pallas import tpu_sc as plsc

NC, NS = 2, 16
NW = NC * NS
NBUF = 2
LANES = 16


def kernel(x, pos_table):
    B, S, D = x.shape
    seq_per_w = S // NW
    vregs_per_row = D // LANES
    mesh = plsc.VectorSubcoreMesh(core_axis_name="c", subcore_axis_name="s")

    @functools.partial(
        pl.kernel,
        out_type=jax.ShapeDtypeStruct((B, S, D), jnp.float32),
        mesh=mesh,
        scratch_types=[
            pltpu.VMEM((seq_per_w, D), jnp.float32),
            pltpu.VMEM((NBUF, seq_per_w, D), jnp.float32),
            pltpu.SemaphoreType.DMA,
            pltpu.SemaphoreType.DMA((NBUF,)),
            pltpu.SemaphoreType.DMA((NBUF,)),
        ],
    )
    def sc_add(x_hbm, pos_hbm, out_hbm, pos_v, bufs, sem_p, sem_in, sem_out):
        wid = lax.axis_index("s") * NC + lax.axis_index("c")
        s0 = wid * seq_per_w
        pos_desc = pltpu.async_copy(pos_hbm.at[pl.ds(s0, seq_per_w)], pos_v, sem_p)

        def in_copy(c):
            b = c % NBUF
            return pltpu.async_copy(
                x_hbm.at[c, pl.ds(s0, seq_per_w)], bufs.at[b], sem_in.at[b])

        def out_copy(c):
            b = c % NBUF
            return pltpu.async_copy(
                bufs.at[b], out_hbm.at[c, pl.ds(s0, seq_per_w)], sem_out.at[b])

        descs = {("in", 0): in_copy(0)}
        pos_desc.wait()
        for c in range(B):
            b = c % NBUF
            if c + 1 < B:
                if c >= 1:
                    descs[("out", c - 1)].wait()
                descs[("in", c + 1)] = in_copy(c + 1)
            descs[("in", c)].wait()

            rows_per_it = 8
            def add_rows(g, _, b=b):
                r0 = g * rows_per_it
                for dr in range(rows_per_it):
                    for k in range(vregs_per_row):
                        plsc.addupdate(
                            bufs.at[b, r0 + dr, pl.ds(k * LANES, LANES)],
                            pos_v[r0 + dr, pl.ds(k * LANES, LANES)])
                return 0
            lax.fori_loop(0, seq_per_w // rows_per_it, add_rows, 0)

            descs[("out", c)] = out_copy(c)
        descs[("out", B - 2)].wait()
        descs[("out", B - 1)].wait()

    return sc_add(x, pos_table)
